# baseline (speedup 1.0000x reference)
import dataclasses
import functools

import jax
import jax.numpy as jnp
from jax.experimental import pallas as pl
from jax.experimental.pallas import tpu as pltpu
from jax.experimental.pallas import tpu_sc as plsc

QK_DIM = 64
TOPK = 16
SCALE = QK_DIM ** (-0.5)
SEQ = 1024


def _logits_kernel(q_ref, k_ref, o_ref):
    o_ref[0] = jax.lax.dot_general(
        q_ref[0] * SCALE,
        k_ref[0],
        (((1,), (1,)), ((), ())),
        preferred_element_type=jnp.float32,
    )


def _logits(q, k):
    n = q.shape[0]
    return pl.pallas_call(
        _logits_kernel,
        grid=(n,),
        in_specs=[
            pl.BlockSpec((1, SEQ, QK_DIM), lambda b: (b, 0, 0)),
            pl.BlockSpec((1, SEQ, QK_DIM), lambda b: (b, 0, 0)),
        ],
        out_specs=pl.BlockSpec((1, SEQ, SEQ), lambda b: (b, 0, 0)),
        out_shape=jax.ShapeDtypeStruct((n, SEQ, SEQ), jnp.float32),
    )(q, k)


def _sc_topk(x):
    rows = x.shape[0]
    mesh = plsc.VectorSubcoreMesh(core_axis_name="c", subcore_axis_name="s")

    cp = pltpu.CompilerParams()
    if "needs_layout_passes" in pltpu.CompilerParams.__dataclass_fields__:
        cp = dataclasses.replace(cp, needs_layout_passes=False)

    @pl.kernel(
        out_type=[
            jax.ShapeDtypeStruct((rows, TOPK), jnp.float32),
            jax.ShapeDtypeStruct((rows, TOPK), jnp.int32),
        ],
        mesh=mesh,
        compiler_params=cp,
    )
    def sck(x_hbm, w_hbm, i_hbm):
        def body(x_vmem, w_vmem, i_vmem):
            xr = x_vmem.at[0]
            cur_v, cur_i = plsc.sort_key_val(
                xr[pl.ds(0, TOPK)], jax.lax.iota(jnp.int32, TOPK),
                descending=True,
            )
            for ch in range(1, SEQ // TOPK):
                v = xr[pl.ds(ch * TOPK, TOPK)]
                ci = jax.lax.iota(jnp.int32, TOPK) + ch * TOPK

                def _merge(cur_v=cur_v, cur_i=cur_i, v=v, ci=ci):
                    sv, si = plsc.sort_key_val(v, ci)
                    mv = jnp.maximum(cur_v, sv)
                    mi = jnp.where(cur_v >= sv, cur_i, si)
                    nv, ni = plsc.sort_key_val(mv, mi, descending=True)
                    return nv, ni

                def _skip(cur_v=cur_v, cur_i=cur_i):
                    return cur_v, cur_i

                cur_v, cur_i = jax.lax.cond(
                    jnp.max(v) > jnp.min(cur_v), _merge, _skip
                )
            m = jnp.max(cur_v)
            e = jnp.exp(cur_v - m)
            w_vmem[0, :] = e / jnp.sum(e)
            i_vmem[0, :] = cur_i

        pltpu.emit_pipeline(
            body,
            grid=(rows,),
            in_specs=[pl.BlockSpec((1, SEQ), lambda r: (r, 0))],
            out_specs=[
                pl.BlockSpec((1, TOPK), lambda r: (r, 0)),
                pl.BlockSpec((1, TOPK), lambda r: (r, 0)),
            ],
            core_axis_name=("c", "s"),
            dimension_semantics=(pltpu.PARALLEL,),
        )(x_hbm, w_hbm, i_hbm)

    return sck(x)


N_SLICES = 8


@jax.jit
def kernel(query, key):
    n, s, c = query.shape
    step = n // N_SLICES
    ws, ixs = [], []
    for p in range(N_SLICES):
        qp = query[p * step : (p + 1) * step]
        kp = key[p * step : (p + 1) * step]
        logits = _logits(qp, kp)
        w, ix = _sc_topk(logits.reshape(step * s, s))
        ws.append(w.reshape(step, s, TOPK))
        ixs.append(ix.reshape(step, s, TOPK))
    return jnp.concatenate(ws, axis=0), jnp.concatenate(ixs, axis=0)


# speedup vs baseline: 1.9115x; 1.9115x over previous
import dataclasses
import functools

import jax
import jax.numpy as jnp
from jax.experimental import pallas as pl
from jax.experimental.pallas import tpu as pltpu
from jax.experimental.pallas import tpu_sc as plsc

QK_DIM = 64
TOPK = 16
SCALE = QK_DIM ** (-0.5)
SEQ = 1024


def _logits_kernel(q_ref, k_ref, o_ref):
    o_ref[0] = jax.lax.dot_general(
        q_ref[0] * SCALE,
        k_ref[0],
        (((1,), (1,)), ((), ())),
        preferred_element_type=jnp.float32,
    )


def _logits(q, k):
    n = q.shape[0]
    return pl.pallas_call(
        _logits_kernel,
        grid=(n,),
        in_specs=[
            pl.BlockSpec((1, SEQ, QK_DIM), lambda b: (b, 0, 0)),
            pl.BlockSpec((1, SEQ, QK_DIM), lambda b: (b, 0, 0)),
        ],
        out_specs=pl.BlockSpec((1, SEQ, SEQ), lambda b: (b, 0, 0)),
        out_shape=jax.ShapeDtypeStruct((n, SEQ, SEQ), jnp.float32),
    )(q, k)


def _sc_topk(x):
    rows = x.shape[0]
    mesh = plsc.VectorSubcoreMesh(core_axis_name="c", subcore_axis_name="s")

    cp = pltpu.CompilerParams()
    if "needs_layout_passes" in pltpu.CompilerParams.__dataclass_fields__:
        cp = dataclasses.replace(cp, needs_layout_passes=False)

    @pl.kernel(
        out_type=[
            jax.ShapeDtypeStruct((rows, TOPK), jnp.float32),
            jax.ShapeDtypeStruct((rows, TOPK), jnp.int32),
        ],
        mesh=mesh,
        compiler_params=cp,
    )
    def sck(x_hbm, w_hbm, i_hbm):
        def body(x_vmem, w_vmem, i_vmem):
            xr = x_vmem.at[0]
            cur_v, cur_i = plsc.sort_key_val(
                xr[pl.ds(0, TOPK)], jax.lax.iota(jnp.int32, TOPK),
                descending=True,
            )
            for ch in range(1, SEQ // TOPK):
                v = xr[pl.ds(ch * TOPK, TOPK)]
                ci = jax.lax.iota(jnp.int32, TOPK) + ch * TOPK
                sv, si = plsc.sort_key_val(v, ci)
                mv = jnp.maximum(cur_v, sv)
                mi = jnp.where(cur_v >= sv, cur_i, si)
                cur_v, cur_i = plsc.sort_key_val(mv, mi, descending=True)
            m = jnp.max(cur_v)
            e = jnp.exp(cur_v - m)
            w_vmem[0, :] = e / jnp.sum(e)
            i_vmem[0, :] = cur_i

        pltpu.emit_pipeline(
            body,
            grid=(rows,),
            in_specs=[pl.BlockSpec((1, SEQ), lambda r: (r, 0))],
            out_specs=[
                pl.BlockSpec((1, TOPK), lambda r: (r, 0)),
                pl.BlockSpec((1, TOPK), lambda r: (r, 0)),
            ],
            core_axis_name=("c", "s"),
            dimension_semantics=(pltpu.PARALLEL,),
        )(x_hbm, w_hbm, i_hbm)

    return sck(x)


N_SLICES = 4


@jax.jit
def kernel(query, key):
    n, s, c = query.shape
    step = n // N_SLICES
    ws, ixs = [], []
    for p in range(N_SLICES):
        qp = query[p * step : (p + 1) * step]
        kp = key[p * step : (p + 1) * step]
        logits = _logits(qp, kp)
        w, ix = _sc_topk(logits.reshape(step * s, s))
        ws.append(w.reshape(step, s, TOPK))
        ixs.append(ix.reshape(step, s, TOPK))
    return jnp.concatenate(ws, axis=0), jnp.concatenate(ixs, axis=0)
